# Initial kernel scaffold; baseline (speedup 1.0000x reference)
import jax
import jax.numpy as jnp
from jax.experimental import pallas as pl


def kernel(edge_index, feat, shuf_feat, lin_w, lin_b, temp, bil_w, bil_b, alpha, beta):
    raise NotImplementedError("write your pallas kernel here")


# speedup vs baseline: 1.0725x; 1.0725x over previous
import math
import functools

import jax
import jax.numpy as jnp
import numpy as np
from jax.experimental import pallas as pl
from jax.experimental.pallas import tpu as pltpu

_K = 10


def _cheb_mat(k):
    xs = np.array([math.cos((k - j + 0.5) * math.pi / (k + 1)) for j in range(k + 1)],
                  dtype=np.float64)
    C = np.zeros((k + 1, k + 1), dtype=np.float64)
    C[0, :] = 1.0
    C[1, :] = xs
    for i in range(2, k + 1):
        C[i, :] = 2.0 * xs * C[i - 1, :] - C[i - 2, :]
    return C


_CMAT = jnp.asarray(_cheb_mat(_K), dtype=jnp.float32)


def _mm_body(x_ref, w_ref, b_ref, o_ref):
    acc = jnp.dot(x_ref[...], w_ref[...], preferred_element_type=jnp.float32)
    o_ref[...] = jnp.maximum(acc + b_ref[...], 0.0)


def _linear_relu(x, w_t, b):
    R = x.shape[0]
    BR = 1000
    grid = (R // BR,)
    return pl.pallas_call(
        _mm_body,
        grid=grid,
        in_specs=[
            pl.BlockSpec((BR, x.shape[1]), lambda i: (i, 0)),
            pl.BlockSpec((x.shape[1], w_t.shape[1]), lambda i: (0, 0)),
            pl.BlockSpec((1, w_t.shape[1]), lambda i: (0, 0)),
        ],
        out_specs=pl.BlockSpec((BR, w_t.shape[1]), lambda i: (i, 0)),
        out_shape=jax.ShapeDtypeStruct((R, w_t.shape[1]), jnp.float32),
    )(x, w_t, b.reshape(1, -1))


def kernel(edge_index, feat, shuf_feat, lin_w, lin_b, temp, bil_w, bil_b, alpha, beta):
    N, D = feat.shape
    src = edge_index[0]
    dst = edge_index[1]
    deg = jnp.zeros((N,), feat.dtype).at[src].add(1.0)
    dinv = jnp.where(deg > 0, 1.0 / jnp.sqrt(jnp.maximum(deg, 1e-12)), 0.0)

    coe_hi = (2.0 / (_K + 1)) * (_CMAT @ jax.nn.relu(temp))
    coe_lo = (2.0 / (_K + 1)) * (_CMAT @ jax.nn.relu(jnp.flip(temp, axis=0)))

    dcol = dinv[:, None]

    def seq(x):
        Tx0 = x
        acc = jnp.zeros_like(x).at[dst].add(jnp.take(dcol * Tx0, src, axis=0))
        Tx1 = -dcol * acc
        oh = (coe_hi[0] / 2.0) * Tx0 + coe_hi[1] * Tx1
        ol = (coe_lo[0] / 2.0) * Tx0 + coe_lo[1] * Tx1
        for i in range(2, _K + 1):
            acc = jnp.zeros_like(x).at[dst].add(jnp.take(dcol * Tx1, src, axis=0))
            Tx2 = -2.0 * dcol * acc - Tx0
            oh = oh + coe_hi[i] * Tx2
            ol = ol + coe_lo[i] * Tx2
            Tx0, Tx1 = Tx1, Tx2
        return oh, ol

    oh_f, ol_f = seq(feat)
    oh_s, ol_s = seq(shuf_feat)
    outs = jnp.concatenate([oh_f, ol_f, oh_s, ol_s], axis=0)

    h_all = _linear_relu(outs, lin_w.T, lin_b)
    h1 = h_all[0 * N:1 * N]
    h2 = h_all[1 * N:2 * N]
    h3 = h_all[2 * N:3 * N]
    h4 = h_all[3 * N:4 * N]

    h = alpha * h1 + beta * h2
    c = jax.nn.relu(jnp.mean(h, axis=0))
    v = bil_w[0] @ c

    sc_1 = h2 @ v + bil_b[0]
    sc_2 = h1 @ v + bil_b[0]
    sc_3 = h4 @ v + bil_b[0]
    sc_4 = h3 @ v + bil_b[0]
    return jnp.concatenate([sc_1, sc_2, sc_3, sc_4], axis=0)


# speedup vs baseline: 2.8316x; 2.6402x over previous
import math
import functools

import jax
import jax.numpy as jnp
import numpy as np
from jax import lax
from jax.experimental import pallas as pl
from jax.experimental.pallas import tpu as pltpu
from jax.experimental.pallas import tpu_sc as plsc

_K = 10
_N = 10000
_E = 160000
_NP = 10240
_PT = _NP // 16
_CHUNK = 128
_EPT = 10240
_NCH = _EPT // _CHUNK
_NBUF = 4
_DB = 64
_NBLK = 8


def _cheb_mat(k):
    xs = np.array([math.cos((k - j + 0.5) * math.pi / (k + 1)) for j in range(k + 1)],
                  dtype=np.float64)
    C = np.zeros((k + 1, k + 1), dtype=np.float64)
    C[0, :] = 1.0
    C[1, :] = xs
    for i in range(2, k + 1):
        C[i, :] = 2.0 * xs * C[i - 1, :] - C[i - 2, :]
    return C


_CMAT = np.asarray(_cheb_mat(_K), dtype=np.float32)

_SC_MESH = plsc.VectorSubcoreMesh(
    core_axis_name="c", subcore_axis_name="s", num_cores=2, num_subcores=16)


def _sc_deg_body(src_hbm, ones_hbm, z16_hbm, d0_hbm, d1_hbm,
                 sidx, ones_v, acc, sem):
    c = lax.axis_index("c")
    s = lax.axis_index("s")
    pltpu.sync_copy(src_hbm.at[pl.ds(s * _NCH, _NCH)], sidx)
    pltpu.sync_copy(ones_hbm, ones_v)
    pltpu.sync_copy(z16_hbm, acc.at[pl.ds(s * _PT, _PT)])
    plsc.subcore_barrier()

    def step(j, _):
        pltpu.sync_copy(ones_v, acc.at[sidx.at[j]], add=True)
        return 0
    lax.fori_loop(c * (_NCH // 2), c * (_NCH // 2) + _NCH // 2, step, 0)
    plsc.subcore_barrier()

    @pl.when(c == 0)
    def _():
        pltpu.sync_copy(acc.at[pl.ds(s * _PT, _PT)], d0_hbm.at[pl.ds(s * _PT, _PT)])

    @pl.when(c == 1)
    def _():
        pltpu.sync_copy(acc.at[pl.ds(s * _PT, _PT)], d1_hbm.at[pl.ds(s * _PT, _PT)])


_sc_degree = pl.kernel(
    _sc_deg_body,
    out_type=(jax.ShapeDtypeStruct((_NP, 16), jnp.float32),
              jax.ShapeDtypeStruct((_NP, 16), jnp.float32)),
    mesh=_SC_MESH,
    scratch_types=[
        pltpu.VMEM((_NCH, _CHUNK), jnp.int32),
        pltpu.VMEM((_CHUNK, 16), jnp.float32),
        pltpu.VMEM_SHARED((_NP, 16), jnp.float32),
        pltpu.SemaphoreType.DMA,
    ],
    compiler_params=pltpu.CompilerParams(use_tc_tiling_on_sc=False),
)


def _sc_hop_body(y_hbm, src8_hbm, dst_hbm, z640_hbm, out_hbm,
                 sidx, didx, r0, r1, r2, r3,
                 acc, sem0, sem1, sem2, sem3):
    c = lax.axis_index("c")
    s = lax.axis_index("s")
    rows = [r0, r1, r2, r3]
    sems = [sem0, sem1, sem2, sem3]
    dummy = z640_hbm.at[pl.ds(0, _CHUNK)]

    pltpu.sync_copy(dst_hbm.at[pl.ds(s * _NCH, _NCH)], didx)

    for cc in (0, 1):
        @pl.when(c == cc)
        def _(cc=cc):
            for bl in range(_NBLK // 2):
                b = (_NBLK // 2) * cc + bl
                pltpu.sync_copy(
                    src8_hbm.at[pl.ds((b * 16 + s) * _NCH, _NCH)], sidx)
                pltpu.sync_copy(z640_hbm, acc.at[pl.ds(s * _PT, _PT)])
                plsc.subcore_barrier()

                for bb in range(_NBUF):
                    pltpu.async_copy(y_hbm.at[sidx.at[bb]], rows[bb], sems[bb])

                def step(g4, _):
                    g = g4 * _NBUF
                    for bb in range(_NBUF):
                        j = g + bb
                        pltpu.make_async_copy(dummy, rows[bb], sems[bb]).wait()
                        pltpu.sync_copy(rows[bb], acc.at[didx.at[j]], add=True)

                        @pl.when(j + _NBUF < _NCH)
                        def _(bb=bb, j=j):
                            pltpu.async_copy(
                                y_hbm.at[sidx.at[j + _NBUF]], rows[bb], sems[bb])
                    return 0
                lax.fori_loop(0, _NCH // _NBUF, step, 0)
                plsc.subcore_barrier()
                pltpu.sync_copy(acc.at[pl.ds(s * _PT, _PT)],
                                out_hbm.at[pl.ds(b * _NP + s * _PT, _PT)])
                plsc.subcore_barrier()


_sc_hop = pl.kernel(
    _sc_hop_body,
    out_type=jax.ShapeDtypeStruct((_NBLK * _NP, _DB), jnp.float32),
    mesh=_SC_MESH,
    scratch_types=[
        pltpu.VMEM((_NCH, _CHUNK), jnp.int32),
        pltpu.VMEM((_NCH, _CHUNK), jnp.int32),
        pltpu.VMEM((_CHUNK, _DB), jnp.float32),
        pltpu.VMEM((_CHUNK, _DB), jnp.float32),
        pltpu.VMEM((_CHUNK, _DB), jnp.float32),
        pltpu.VMEM((_CHUNK, _DB), jnp.float32),
        pltpu.VMEM_SHARED((_NP, _DB), jnp.float32),
        pltpu.SemaphoreType.DMA,
        pltpu.SemaphoreType.DMA,
        pltpu.SemaphoreType.DMA,
        pltpu.SemaphoreType.DMA,
    ],
    compiler_params=pltpu.CompilerParams(use_tc_tiling_on_sc=False),
)


_BRN = 2560


def _dinv_blk(d0_ref, d1_ref):
    deg = d0_ref[:, 0:1] + d1_ref[:, 0:1]
    return jnp.where(deg > 0, lax.rsqrt(jnp.maximum(deg, 1e-12)), 0.0)


def _init_body(d0_ref, d1_ref, x_ref, y_ref):
    y_ref[...] = _dinv_blk(d0_ref, d1_ref) * x_ref[...]


def _hop1_body(d0_ref, d1_ref, acc_ref, x_ref, ch_ref, cl_ref,
               tx1_ref, y_ref, oh_ref, ol_ref):
    dinv = _dinv_blk(d0_ref, d1_ref)
    tx1 = -dinv * acc_ref[...]
    x = x_ref[...]
    tx1_ref[...] = tx1
    y_ref[...] = dinv * tx1
    oh_ref[...] = (ch_ref[0, 0] * 0.5) * x + ch_ref[0, 1] * tx1
    ol_ref[...] = (cl_ref[0, 0] * 0.5) * x + cl_ref[0, 1] * tx1


def _mk_hop_body(i):
    def _hop_body(d0_ref, d1_ref, acc_ref, tx0_ref, oh_ref, ol_ref,
                  ch_ref, cl_ref, tx2_ref, y_ref, oh2_ref, ol2_ref):
        dinv = _dinv_blk(d0_ref, d1_ref)
        tx2 = -2.0 * dinv * acc_ref[...] - tx0_ref[...]
        tx2_ref[...] = tx2
        y_ref[...] = dinv * tx2
        oh2_ref[...] = oh_ref[...] + ch_ref[0, i] * tx2
        ol2_ref[...] = ol_ref[...] + cl_ref[0, i] * tx2
    return _hop_body


_BLK = pl.BlockSpec((_BRN, _DB), lambda i: (i, 0))
_DBLK = pl.BlockSpec((_BRN, 16), lambda i: (i % 4, 0))
_CBLK = pl.BlockSpec((1, 16), lambda i: (0, 0))
_G32 = (_NBLK * _NP // _BRN,)
_F = jax.ShapeDtypeStruct((_NBLK * _NP, _DB), jnp.float32)


def _tc_init(d0, d1, xcat):
    return pl.pallas_call(
        _init_body, grid=_G32,
        in_specs=[_DBLK, _DBLK, _BLK], out_specs=_BLK, out_shape=_F,
    )(d0, d1, xcat)


def _tc_hop1(d0, d1, acc, xcat, ch, cl):
    return pl.pallas_call(
        _hop1_body, grid=_G32,
        in_specs=[_DBLK, _DBLK, _BLK, _BLK, _CBLK, _CBLK],
        out_specs=(_BLK, _BLK, _BLK, _BLK),
        out_shape=(_F, _F, _F, _F),
    )(d0, d1, acc, xcat, ch, cl)


def _tc_hop(i, d0, d1, acc, tx0, oh, ol, ch, cl):
    return pl.pallas_call(
        _mk_hop_body(i), grid=_G32,
        in_specs=[_DBLK, _DBLK, _BLK, _BLK, _BLK, _BLK, _CBLK, _CBLK],
        out_specs=(_BLK, _BLK, _BLK, _BLK),
        out_shape=(_F, _F, _F, _F),
        input_output_aliases={4: 2, 5: 3},
    )(d0, d1, acc, tx0, oh, ol, ch, cl)


def _mm_body(x_ref, w_ref, b_ref, o_ref):
    acc = jnp.dot(x_ref[...], w_ref[...], preferred_element_type=jnp.float32)
    o_ref[...] = jnp.maximum(acc + b_ref[...], 0.0)


def _linear_relu(x, w_t, b):
    R = x.shape[0]
    BR = 1000
    grid = (R // BR,)
    return pl.pallas_call(
        _mm_body,
        grid=grid,
        in_specs=[
            pl.BlockSpec((BR, x.shape[1]), lambda i: (i, 0)),
            pl.BlockSpec((x.shape[1], w_t.shape[1]), lambda i: (0, 0)),
            pl.BlockSpec((1, w_t.shape[1]), lambda i: (0, 0)),
        ],
        out_specs=pl.BlockSpec((BR, w_t.shape[1]), lambda i: (i, 0)),
        out_shape=jax.ShapeDtypeStruct((R, w_t.shape[1]), jnp.float32),
    )(x, w_t, b.reshape(1, -1))


def _cv_body(h_ref, ab_ref, w0t_ref, v_ref, s_acc):
    i = pl.program_id(0)

    @pl.when(i == 0)
    def _():
        s_acc[...] = jnp.zeros_like(s_acc)

    scale = jnp.where(i < 5, ab_ref[0, 0], ab_ref[0, 1])
    part = jnp.sum(h_ref[...], axis=0, keepdims=True)
    s_acc[...] += scale * jnp.broadcast_to(part, s_acc.shape)

    @pl.when(i == 9)
    def _():
        c = jnp.maximum(s_acc[0:1, :] * (1.0 / _N), 0.0)
        v = jnp.dot(c, w0t_ref[...], preferred_element_type=jnp.float32)
        v_ref[...] = jnp.broadcast_to(v, v_ref.shape)


def _tc_cv(h12, ab, w0t):
    BR = 2000
    return pl.pallas_call(
        _cv_body, grid=(2 * _N // BR,),
        in_specs=[
            pl.BlockSpec((BR, 512), lambda i: (i, 0)),
            pl.BlockSpec((1, 2), lambda i: (0, 0)),
            pl.BlockSpec((512, 512), lambda i: (0, 0)),
        ],
        out_specs=pl.BlockSpec((8, 512), lambda i: (0, 0)),
        out_shape=jax.ShapeDtypeStruct((8, 512), jnp.float32),
        scratch_shapes=[pltpu.VMEM((8, 512), jnp.float32)],
    )(h12, ab, w0t)


def _score_body(h_ref, v_ref, bb_ref, o_ref):
    o_ref[...] = jnp.dot(h_ref[...], v_ref[...],
                         preferred_element_type=jnp.float32) + bb_ref[0, 0]


def _tc_scores(h_ord, vcol, bb):
    BR = 2000
    return pl.pallas_call(
        _score_body, grid=(4 * _N // BR,),
        in_specs=[
            pl.BlockSpec((BR, 512), lambda i: (i, 0)),
            pl.BlockSpec((512, 1), lambda i: (0, 0)),
            pl.BlockSpec((1, 1), lambda i: (0, 0)),
        ],
        out_specs=pl.BlockSpec((BR, 1), lambda i: (i, 0)),
        out_shape=jax.ShapeDtypeStruct((4 * _N, 1), jnp.float32),
    )(h_ord, vcol, bb)


def kernel(edge_index, feat, shuf_feat, lin_w, lin_b, temp, bil_w, bil_b, alpha, beta):
    f32 = jnp.float32

    pad_e = 16 * _EPT - _E
    src = jnp.concatenate([edge_index[0], jnp.full((pad_e,), _N, jnp.int32)])
    dst = jnp.concatenate([edge_index[1], jnp.full((pad_e,), _N, jnp.int32)])
    dst2 = dst.reshape(16 * _NCH, _CHUNK)
    src2 = src.reshape(16 * _NCH, _CHUNK)
    src8 = (src2[None, :, :] +
            (jnp.arange(_NBLK, dtype=jnp.int32) * _NP)[:, None, None]
            ).reshape(_NBLK * 16 * _NCH, _CHUNK)

    ones16 = jnp.ones((_CHUNK, 16), f32)
    z16 = jnp.zeros((_PT, 16), f32)
    z640 = jnp.zeros((_PT, _DB), f32)

    fp = jnp.pad(feat, ((0, _NP - _N), (0, 0)))
    sp = jnp.pad(shuf_feat, ((0, _NP - _N), (0, 0)))
    xcat = jnp.concatenate(
        [fp[:, k * _DB:(k + 1) * _DB] for k in range(4)] +
        [sp[:, k * _DB:(k + 1) * _DB] for k in range(4)], axis=0)

    cmat = jnp.asarray(_CMAT)
    coe_hi = (2.0 / (_K + 1)) * (cmat @ jax.nn.relu(temp))
    coe_lo = (2.0 / (_K + 1)) * (cmat @ jax.nn.relu(jnp.flip(temp, axis=0)))
    ch = jnp.pad(coe_hi, (0, 16 - (_K + 1))).reshape(1, 16)
    cl = jnp.pad(coe_lo, (0, 16 - (_K + 1))).reshape(1, 16)

    d0, d1 = _sc_degree(src2, ones16, z16)

    y = _tc_init(d0, d1, xcat)
    acc = _sc_hop(y, src8, dst2, z640)
    tx1, y, oh, ol = _tc_hop1(d0, d1, acc, xcat, ch, cl)
    tx0 = xcat
    for i in range(2, _K + 1):
        acc = _sc_hop(y, src8, dst2, z640)
        tx2, y, oh, ol = _tc_hop(i, d0, d1, acc, tx0, oh, ol, ch, cl)
        tx0, tx1 = tx1, tx2

    ohb = oh.reshape(_NBLK, _NP, _DB)
    olb = ol.reshape(_NBLK, _NP, _DB)

    def _cols(bb, lo):
        return jnp.concatenate([bb[k, :_N] for k in range(lo, lo + 4)], axis=1)

    outs = jnp.concatenate([
        _cols(ohb, 0), _cols(olb, 0), _cols(ohb, 4), _cols(olb, 4),
    ], axis=0)

    h_all = _linear_relu(outs, lin_w.T, lin_b)

    ab = jnp.stack([alpha, beta]).astype(f32).reshape(1, 2)
    v8 = _tc_cv(h_all[:2 * _N], ab, bil_w[0].T)
    vcol = v8[0].reshape(512, 1)

    h_ord = jnp.concatenate([
        h_all[_N:2 * _N], h_all[:_N], h_all[3 * _N:], h_all[2 * _N:3 * _N]])
    scores = _tc_scores(h_ord, vcol, bil_b.reshape(1, 1))
    return scores[:, 0]
